# baseline (speedup 1.0000x reference)
import functools

import jax
import jax.numpy as jnp
from jax import lax
from jax.experimental import pallas as pl
from jax.experimental.pallas import tpu as pltpu
from jax.experimental.pallas import tpu_sc as plsc

NUM_TABLES = 26
VOCAB = 100000
DIM = 64
BATCH = 4096
L = 20

NC = 2
NS = 16
NW = NC * NS
LANES = 16
ROWP = 2 * DIM

BAGS_PER_W = BATCH // NW
CHUNK = 8
BLOCKS = BAGS_PER_W // CHUNK
ROWS_PER_CHUNK = 2 * CHUNK * L
IDX_ROWS = 3
GSIZES = (128, 128, 64)

GROUP_PAIRS = (4, 3, 3, 3)


def _make_emb_kernel(pairs_g):
  n_chunks = pairs_g * BLOCKS

  def body(idx_hbm, tbl_hbm, out_hbm,
           idx0, idx1, rows0, rows1, ob0, ob1,
           isem0, isem1, gsem0, gsem1, osem0, osem1):
    w = lax.axis_index("s") * NC + lax.axis_index("c")

    def idx_cp(i, ib, sem):
      return pltpu.make_async_copy(idx_hbm.at[w * n_chunks + i], ib, sem)

    def gath(ib, rb, sem, j):
      sz = GSIZES[j]
      return pltpu.make_async_copy(
          tbl_hbm.at[ib.at[j, pl.ds(0, sz)]], rb.at[pl.ds(j * 128, sz)], sem)

    def out_cp(i, ob, sem):
      p = i // BLOCKS
      c = i % BLOCKS
      b0 = w * BAGS_PER_W + c * CHUNK
      return pltpu.make_async_copy(
          ob, out_hbm.at[pl.ds(b0, CHUNK), pl.ds(p * ROWP, ROWP)], sem)

    def accumulate(rb, ob):
      def bag(c, carry):
        for h in range(2):
          base = h * (CHUNK * L) + c * L
          for d in range(DIM // LANES):
            acc = rb[base, pl.ds(d * LANES, LANES)]
            for l in range(1, L):
              acc = acc + rb[base + l, pl.ds(d * LANES, LANES)]
            ob[c, pl.ds(h * DIM + d * LANES, LANES)] = acc
        return carry
      lax.fori_loop(0, CHUNK, bag, 0)

    idx_cp(0, idx0, isem0).start()
    idx_cp(0, idx0, isem0).wait()
    for j in range(len(GSIZES)):
      gath(idx0, rows0, gsem0, j).start()
    idx_cp(1, idx1, isem1).start()

    def step(i2, carry):
      i = i2 * 2

      idx_cp(i + 1, idx1, isem1).wait()
      for j in range(len(GSIZES)):
        gath(idx1, rows1, gsem1, j).start()
      for j in range(len(GSIZES)):
        gath(idx0, rows0, gsem0, j).wait()

      @pl.when(i + 2 < n_chunks)
      def _():
        idx_cp(i + 2, idx0, isem0).start()

      @pl.when(i >= 2)
      def _():
        out_cp(i - 2, ob0, osem0).wait()

      accumulate(rows0, ob0)
      out_cp(i, ob0, osem0).start()

      @pl.when(i + 2 < n_chunks)
      def _():
        idx_cp(i + 2, idx0, isem0).wait()
        for j in range(len(GSIZES)):
          gath(idx0, rows0, gsem0, j).start()

      for j in range(len(GSIZES)):
        gath(idx1, rows1, gsem1, j).wait()

      @pl.when(i + 3 < n_chunks)
      def _():
        idx_cp(i + 3, idx1, isem1).start()

      @pl.when(i >= 2)
      def _():
        out_cp(i - 1, ob1, osem1).wait()

      accumulate(rows1, ob1)
      out_cp(i + 1, ob1, osem1).start()
      return carry

    lax.fori_loop(0, n_chunks // 2, step, 0)

    out_cp(n_chunks - 2, ob0, osem0).wait()
    out_cp(n_chunks - 1, ob1, osem1).wait()

  return pl.kernel(
      body,
      out_type=jax.ShapeDtypeStruct((BATCH, pairs_g * ROWP), jnp.float32),
      mesh=plsc.VectorSubcoreMesh(
          core_axis_name="c", subcore_axis_name="s",
          num_cores=NC, num_subcores=NS),
      scratch_types=[
          pltpu.VMEM((IDX_ROWS, 128), jnp.int32),
          pltpu.VMEM((IDX_ROWS, 128), jnp.int32),
          pltpu.VMEM((ROWS_PER_CHUNK, ROWP), jnp.float32),
          pltpu.VMEM((ROWS_PER_CHUNK, ROWP), jnp.float32),
          pltpu.VMEM((CHUNK, ROWP), jnp.float32),
          pltpu.VMEM((CHUNK, ROWP), jnp.float32),
          pltpu.SemaphoreType.DMA,
          pltpu.SemaphoreType.DMA,
          pltpu.SemaphoreType.DMA,
          pltpu.SemaphoreType.DMA,
          pltpu.SemaphoreType.DMA,
          pltpu.SemaphoreType.DMA,
      ],
  )


_EMB_KERNELS = {p: _make_emb_kernel(p) for p in set(GROUP_PAIRS)}


def _prep_idx(sub_idx, pairs_g):
  offs = (jnp.arange(2 * pairs_g, dtype=jnp.int32) * VOCAB)[:, None, None]
  idx = sub_idx.astype(jnp.int32) + offs
  idx = idx.reshape(pairs_g, 2, NW, BLOCKS, CHUNK, L)
  idx = idx.transpose(2, 0, 3, 1, 4, 5)
  idx = idx.reshape(NW * pairs_g * BLOCKS, ROWS_PER_CHUNK)
  idx = jnp.pad(idx, ((0, 0), (0, IDX_ROWS * 128 - ROWS_PER_CHUNK)))
  return idx.reshape(NW * pairs_g * BLOCKS, IDX_ROWS, 128)


@jax.jit
def kernel(indices, tables):
  outs = []
  t0 = 0
  for pairs_g in GROUP_PAIRS:
    t1 = t0 + 2 * pairs_g
    idx_g = _prep_idx(indices[t0:t1], pairs_g)
    tbl_g = jnp.pad(tables[t0:t1].reshape((t1 - t0) * VOCAB, DIM),
                    ((0, 0), (0, DIM)))
    outs.append(_EMB_KERNELS[pairs_g](idx_g, tbl_g))
    t0 = t1
  return jnp.concatenate(outs, axis=1)
